# Initial kernel scaffold; baseline (speedup 1.0000x reference)
import jax
import jax.numpy as jnp
from jax.experimental import pallas as pl


def kernel(target, permutation):
    raise NotImplementedError("write your pallas kernel here")


# speedup vs baseline: 1.0284x; 1.0284x over previous
import dataclasses

import jax
import jax.numpy as jnp
from jax.experimental import pallas as pl
from jax.experimental.pallas import tpu as pltpu
from jax.experimental.pallas import tpu_sc as plsc

_COMPILER_PARAMS = pltpu.CompilerParams()
if "needs_layout_passes" in pltpu.CompilerParams.__dataclass_fields__:
    _COMPILER_PARAMS = dataclasses.replace(
        _COMPILER_PARAMS, needs_layout_passes=False
    )

D = 2048
ROWS = 4 * 4096
RB = 8
LANES = 16


@jax.jit
def _permute_rows(flat, perm):
    mesh = plsc.VectorSubcoreMesh(core_axis_name="core",
                                  subcore_axis_name="subcore")

    @pl.kernel(
        out_type=jax.ShapeDtypeStruct((ROWS, D), jnp.float32),
        mesh=mesh,
        compiler_params=_COMPILER_PARAMS,
        scratch_types=[
            pltpu.VMEM((D,), jnp.int32),
            pltpu.SemaphoreType.DMA,
        ],
    )
    def kern(x_hbm, p_hbm, o_hbm, perm_vmem, sem):
        pltpu.async_copy(p_hbm, perm_vmem, sem).wait()

        def body(in_vmem, out_vmem):
            @pl.loop(0, D // LANES)
            def _col(j):
                idx = perm_vmem[pl.ds(j * LANES, LANES)]

                @pl.loop(0, RB)
                def _row(r):
                    r_vec = jnp.full((LANES,), r, jnp.int32)
                    out_vmem[r, pl.ds(j * LANES, LANES)] = plsc.load_gather(
                        in_vmem, [r_vec, idx]
                    )

        pltpu.emit_pipeline(
            body,
            grid=(ROWS // RB,),
            in_specs=[pl.BlockSpec((RB, D), index_map=lambda i: (i, 0))],
            out_specs=[pl.BlockSpec((RB, D), index_map=lambda i: (i, 0))],
            core_axis_name=("core", "subcore"),
            dimension_semantics=(pltpu.PARALLEL,),
        )(x_hbm, o_hbm)

    return kern(flat, perm)


def kernel(target, permutation):
    b, s, d = target.shape
    out = _permute_rows(target.reshape(b * s, d), permutation)
    return out.reshape(b, s, d)


# speedup vs baseline: 1.1225x; 1.0914x over previous
import dataclasses

import jax
import jax.numpy as jnp
from jax.experimental import pallas as pl
from jax.experimental.pallas import tpu as pltpu
from jax.experimental.pallas import tpu_sc as plsc

_COMPILER_PARAMS = pltpu.CompilerParams()
if "needs_layout_passes" in pltpu.CompilerParams.__dataclass_fields__:
    _COMPILER_PARAMS = dataclasses.replace(
        _COMPILER_PARAMS, needs_layout_passes=False
    )

D = 2048
ROWS = 4 * 4096
RB = 8
LANES = 16


@jax.jit
def _permute_rows(flat, perm):
    mesh = plsc.VectorSubcoreMesh(core_axis_name="core",
                                  subcore_axis_name="subcore")

    @pl.kernel(
        out_type=jax.ShapeDtypeStruct((ROWS, D), jnp.float32),
        mesh=mesh,
        compiler_params=_COMPILER_PARAMS,
        scratch_types=[
            pltpu.VMEM((D,), jnp.int32),
            pltpu.SemaphoreType.DMA,
        ],
    )
    def kern(x_hbm, p_hbm, o_hbm, perm_vmem, sem):
        pltpu.async_copy(p_hbm, perm_vmem, sem).wait()

        def body(in_vmem, out_vmem):
            @pl.loop(0, D // LANES)
            def _col(j):
                base = j * LANES
                idx = perm_vmem[pl.ds(base, LANES)]
                for r in range(RB):
                    r_vec = jnp.full((LANES,), r, jnp.int32)
                    out_vmem[r, pl.ds(base, LANES)] = plsc.load_gather(
                        in_vmem, [r_vec, idx]
                    )

        pltpu.emit_pipeline(
            body,
            grid=(ROWS // RB,),
            in_specs=[pl.BlockSpec((RB, D), index_map=lambda i: (i, 0))],
            out_specs=[pl.BlockSpec((RB, D), index_map=lambda i: (i, 0))],
            core_axis_name=("core", "subcore"),
            dimension_semantics=(pltpu.PARALLEL,),
        )(x_hbm, o_hbm)

    return kern(flat, perm)


def kernel(target, permutation):
    b, s, d = target.shape
    out = _permute_rows(target.reshape(b * s, d), permutation)
    return out.reshape(b, s, d)
